# Initial kernel scaffold; baseline (speedup 1.0000x reference)
import jax
import jax.numpy as jnp
from jax.experimental import pallas as pl


def kernel(x, edge_index, W1, att1_src, att1_dst, b1, W2, att2_src, att2_dst, b2):
    raise NotImplementedError("write your pallas kernel here")


# speedup vs baseline: 40.8545x; 40.8545x over previous
import functools

import jax
import jax.numpy as jnp
import numpy as np
from jax import lax
from jax.experimental import pallas as pl
from jax.experimental.pallas import tpu as pltpu
from jax.experimental.pallas import tpu_sc as plsc

N_NODES = 10000
N_EDGES = 320000
D_FEAT = 128
HID = 8
HEADS = 8
N_CLASSES = 16

NW = 32
EPW = N_EDGES // NW
CHUNK = 80
NCHUNK = EPW // CHUNK
NPAD = 10240
ROWS = NPAD // 16

_f32 = jnp.float32


def _dense1_body(x_ref, wt_ref, wa_ref, t1_ref, ad_ref):
    xb = x_ref[...]
    t1_ref[...] = jnp.dot(xb, wt_ref[...], preferred_element_type=_f32)
    ad_ref[...] = jnp.dot(xb, wa_ref[...], preferred_element_type=_f32)


def _mid_body(p_ref, b1_ref, m64_ref, b8_ref, w2t_ref, w2a_ref, t2_ref, ad2_ref):
    p = p_ref[...]
    m = p[0] + p[1]
    num = jnp.dot(m, m64_ref[...], preferred_element_type=_f32)
    den = jnp.dot(m, b8_ref[...], preferred_element_type=_f32)
    h = num / (den + 1e-16) + b1_ref[...]
    h = jnp.where(h > 0, h, jnp.exp(h) - 1.0)
    t2_ref[...] = jnp.dot(h, w2t_ref[...], preferred_element_type=_f32)
    ad2_ref[...] = jnp.dot(h, w2a_ref[...], preferred_element_type=_f32)


def _final_body(p_ref, b2_ref, ma_ref, mb_ref, o_ref):
    p = p_ref[...]
    m = p[0] + p[1]
    num = jnp.dot(m, ma_ref[...], preferred_element_type=_f32)
    den = jnp.dot(m, mb_ref[...], preferred_element_type=_f32)
    o = num / (den + 1e-16) + b2_ref[...]
    mx = jnp.max(o, axis=1, keepdims=True)
    e = jnp.exp(o - mx)
    s = jnp.sum(e, axis=1, keepdims=True)
    o_ref[...] = (o - mx) - jnp.log(s)


def _make_edge_sc(DT, DM, nheads):
    mesh = plsc.VectorSubcoreMesh(core_axis_name="c", subcore_axis_name="s")

    def body(t_hbm, ad_hbm, src_hbm, dst_hbm, z_hbm, out_hbm,
             srcv, dstv, G, D, S, obuf, acc, sem1, sem2):
        c = lax.axis_index("c")
        s = lax.axis_index("s")
        wid = s * 2 + c
        pltpu.sync_copy(z_hbm.at[pl.ds(s * ROWS, ROWS)],
                        acc.at[pl.ds(s * ROWS, ROWS)])
        plsc.subcore_barrier()

        base = wid * EPW
        lane = lax.iota(jnp.int32, 16)

        def chunk(ci, _):
            off = base + ci * CHUNK
            pltpu.sync_copy(src_hbm.at[pl.ds(off, CHUNK)], srcv)
            pltpu.sync_copy(dst_hbm.at[pl.ds(off, CHUNK)], dstv)
            pltpu.async_copy(t_hbm.at[srcv], G, sem1).wait()
            pltpu.async_copy(ad_hbm.at[dstv], D, sem2).wait()

            def edge(i, _):
                ga = G[i, pl.ds(DM, 16)]
                da = D[i, pl.ds(0, 16)]
                t = ga + da
                w = jnp.exp(jnp.maximum(t, 0.2 * t))
                if nheads == 8:
                    w = jnp.where(lane < 8, w, 0.0)
                S[i, pl.ds(DM, 16)] = w
                rowi = jnp.full((16,), i, jnp.int32)
                for j in range(DM // 16):
                    if nheads == 8:
                        pat = DM + 2 * j + (lane >= 8).astype(jnp.int32)
                        cj = plsc.load_gather(S, [rowi, pat])
                    else:
                        cj = w
                    S[i, pl.ds(16 * j, 16)] = cj * G[i, pl.ds(16 * j, 16)]
                return 0

            lax.fori_loop(0, CHUNK, edge, 0)
            pltpu.sync_copy(S, acc.at[dstv], add=True)
            return 0

        lax.fori_loop(0, NCHUNK, chunk, 0)
        plsc.subcore_barrier()
        pltpu.sync_copy(acc.at[pl.ds(s * ROWS, ROWS)], obuf)
        pltpu.sync_copy(obuf, out_hbm.at[c, pl.ds(s * ROWS, ROWS)])

    return pl.kernel(
        body,
        out_type=jax.ShapeDtypeStruct((2, NPAD, DT), _f32),
        mesh=mesh,
        compiler_params=pltpu.CompilerParams(
            use_tc_tiling_on_sc=False, needs_layout_passes=False),
        scratch_types=[
            pltpu.VMEM((CHUNK,), jnp.int32),
            pltpu.VMEM((CHUNK,), jnp.int32),
            pltpu.VMEM((CHUNK, DT), _f32),
            pltpu.VMEM((CHUNK, 16), _f32),
            pltpu.VMEM((CHUNK, DT), _f32),
            pltpu.VMEM((ROWS, DT), _f32),
            pltpu.VMEM_SHARED((NPAD, DT), _f32),
            pltpu.SemaphoreType.DMA,
            pltpu.SemaphoreType.DMA,
        ],
    )


_edge1 = _make_edge_sc(80, 64, 8)
_edge2 = _make_edge_sc(32, 16, 1)


_M64 = np.zeros((80, 64), np.float32)
_M64[:64, :] = np.eye(64, dtype=np.float32)
_B8F = np.zeros((80, 64), np.float32)
for _h in range(8):
    _B8F[64 + _h, 8 * _h:8 * _h + 8] = 1.0
_M16A = np.zeros((32, 16), np.float32)
_M16A[:16, :] = np.eye(16, dtype=np.float32)
_M16B = np.zeros((32, 16), np.float32)
_M16B[16:, :] = np.eye(16, dtype=np.float32)


def kernel(x, edge_index, W1, att1_src, att1_dst, b1, W2, att2_src, att2_dst, b2):
    src = edge_index[0]
    dst = edge_index[1]

    W1r = W1.reshape(D_FEAT, HEADS, HID)
    A1s = jnp.einsum('fhc,hc->fh', W1r, att1_src)
    A1d = jnp.einsum('fhc,hc->fh', W1r, att1_dst)
    z8 = jnp.zeros((D_FEAT, 8), _f32)
    W1T = jnp.concatenate([W1, A1s, z8], axis=1)
    W1A = jnp.concatenate([A1d, z8], axis=1)
    A2s = W2 @ att2_src[0]
    A2d = W2 @ att2_dst[0]
    W2T = jnp.concatenate([W2, jnp.tile(A2s[:, None], (1, 16))], axis=1)
    W2A = jnp.tile(A2d[:, None], (1, 16))

    bn = 1000
    grid = N_NODES // bn

    T1, AD1 = pl.pallas_call(
        _dense1_body,
        grid=(grid,),
        in_specs=[
            pl.BlockSpec((bn, D_FEAT), lambda i: (i, 0)),
            pl.BlockSpec((D_FEAT, 80), lambda i: (0, 0)),
            pl.BlockSpec((D_FEAT, 16), lambda i: (0, 0)),
        ],
        out_specs=[
            pl.BlockSpec((bn, 80), lambda i: (i, 0)),
            pl.BlockSpec((bn, 16), lambda i: (i, 0)),
        ],
        out_shape=[
            jax.ShapeDtypeStruct((N_NODES, 80), _f32),
            jax.ShapeDtypeStruct((N_NODES, 16), _f32),
        ],
    )(x, W1T, W1A)

    z1 = jnp.zeros((NPAD, 80), _f32)
    P1 = _edge1(T1, AD1, src, dst, z1)

    T2, AD2 = pl.pallas_call(
        _mid_body,
        grid=(grid,),
        in_specs=[
            pl.BlockSpec((2, bn, 80), lambda i: (0, i, 0)),
            pl.BlockSpec((1, 64), lambda i: (0, 0)),
            pl.BlockSpec((80, 64), lambda i: (0, 0)),
            pl.BlockSpec((80, 64), lambda i: (0, 0)),
            pl.BlockSpec((64, 32), lambda i: (0, 0)),
            pl.BlockSpec((64, 16), lambda i: (0, 0)),
        ],
        out_specs=[
            pl.BlockSpec((bn, 32), lambda i: (i, 0)),
            pl.BlockSpec((bn, 16), lambda i: (i, 0)),
        ],
        out_shape=[
            jax.ShapeDtypeStruct((N_NODES, 32), _f32),
            jax.ShapeDtypeStruct((N_NODES, 16), _f32),
        ],
    )(P1, b1[None, :], jnp.asarray(_M64), jnp.asarray(_B8F), W2T, W2A)

    z2 = jnp.zeros((NPAD, 32), _f32)
    P2 = _edge2(T2, AD2, src, dst, z2)

    out = pl.pallas_call(
        _final_body,
        grid=(grid,),
        in_specs=[
            pl.BlockSpec((2, bn, 32), lambda i: (0, i, 0)),
            pl.BlockSpec((1, 16), lambda i: (0, 0)),
            pl.BlockSpec((32, 16), lambda i: (0, 0)),
            pl.BlockSpec((32, 16), lambda i: (0, 0)),
        ],
        out_specs=pl.BlockSpec((bn, 16), lambda i: (i, 0)),
        out_shape=jax.ShapeDtypeStruct((N_NODES, 16), _f32),
    )(P2, b2[None, :], jnp.asarray(_M16A), jnp.asarray(_M16B))

    return out


# speedup vs baseline: 81.6093x; 1.9976x over previous
import functools

import jax
import jax.numpy as jnp
import numpy as np
from jax import lax
from jax.experimental import pallas as pl
from jax.experimental.pallas import tpu as pltpu
from jax.experimental.pallas import tpu_sc as plsc

N_NODES = 10000
N_EDGES = 320000
D_FEAT = 128
HID = 8
HEADS = 8
N_CLASSES = 16

NW = 32
EPW = N_EDGES // NW
CHUNK = 80
NCHUNK = EPW // CHUNK
NPAD = 10240
ROWS = NPAD // 16

_f32 = jnp.float32


def _dense1_body(x_ref, wt_ref, wa_ref, t1_ref, ad_ref):
    xb = x_ref[...]
    t1_ref[...] = jnp.dot(xb, wt_ref[...], preferred_element_type=_f32)
    ad_ref[...] = jnp.dot(xb, wa_ref[...], preferred_element_type=_f32)


def _mid_body(p_ref, b1_ref, m64_ref, b8_ref, w2t_ref, w2a_ref, t2_ref, ad2_ref):
    p = p_ref[...]
    m = p[0] + p[1]
    num = jnp.dot(m, m64_ref[...], preferred_element_type=_f32)
    den = jnp.dot(m, b8_ref[...], preferred_element_type=_f32)
    h = num / (den + 1e-16) + b1_ref[...]
    h = jnp.where(h > 0, h, jnp.exp(h) - 1.0)
    t2_ref[...] = jnp.dot(h, w2t_ref[...], preferred_element_type=_f32)
    ad2_ref[...] = jnp.dot(h, w2a_ref[...], preferred_element_type=_f32)


def _final_body(p_ref, b2_ref, ma_ref, mb_ref, o_ref):
    p = p_ref[...]
    m = p[0] + p[1]
    num = jnp.dot(m, ma_ref[...], preferred_element_type=_f32)
    den = jnp.dot(m, mb_ref[...], preferred_element_type=_f32)
    o = num / (den + 1e-16) + b2_ref[...]
    mx = jnp.max(o, axis=1, keepdims=True)
    e = jnp.exp(o - mx)
    s = jnp.sum(e, axis=1, keepdims=True)
    o_ref[...] = (o - mx) - jnp.log(s)


def _make_edge_sc(DT, DM, nheads):
    mesh = plsc.VectorSubcoreMesh(core_axis_name="c", subcore_axis_name="s")

    def body(t_hbm, ad_hbm, src_hbm, dst_hbm, z_hbm, out_hbm,
             siv, div, G, D, S, acc, gs0, gs1, ss0, ss1):
        c = lax.axis_index("c")
        s = lax.axis_index("s")
        wid = s * 2 + c
        pltpu.sync_copy(z_hbm.at[pl.ds(s * ROWS, ROWS)],
                        acc.at[pl.ds(s * ROWS, ROWS)])
        pltpu.sync_copy(src_hbm.at[wid], siv)
        pltpu.sync_copy(dst_hbm.at[wid], div)
        plsc.subcore_barrier()

        lane = lax.iota(jnp.int32, 16)
        gsems = (gs0, gs1)
        ssems = (ss0, ss1)

        def fire_gather(ci, b):
            pltpu.async_copy(t_hbm.at[siv.at[ci]], G.at[b], gsems[b])
            pltpu.async_copy(ad_hbm.at[div.at[ci]], D.at[b], gsems[b])

        def wait_gather(b):
            pltpu.make_async_copy(t_hbm.at[siv.at[0]], G.at[b], gsems[b]).wait()
            pltpu.make_async_copy(ad_hbm.at[div.at[0]], D.at[b], gsems[b]).wait()

        def compute(ci, b):
            Gb = G.at[b]
            Db = D.at[b]
            Sb = S.at[b]

            def edge(i, _):
                ga = Gb[i, pl.ds(DM, 16)]
                da = Db[i, pl.ds(0, 16)]
                t = ga + da
                w = jnp.exp(jnp.maximum(t, 0.2 * t))
                if nheads == 8:
                    w = jnp.where(lane < 8, w, 0.0)
                Sb[i, pl.ds(DM, 16)] = w
                rowi = jnp.full((16,), i, jnp.int32)
                for j in range(DM // 16):
                    if nheads == 8:
                        pat = DM + 2 * j + (lane >= 8).astype(jnp.int32)
                        cj = plsc.load_gather(Sb, [rowi, pat])
                    else:
                        cj = w
                    Sb[i, pl.ds(16 * j, 16)] = cj * Gb[i, pl.ds(16 * j, 16)]
                return 0

            lax.fori_loop(0, CHUNK, edge, 0)
            pltpu.async_copy(S.at[b], acc.at[div.at[ci]], ssems[b], add=True)

        def wait_scatter(b):
            pltpu.make_async_copy(z_hbm.at[pl.ds(0, CHUNK)], S.at[b],
                                  ssems[b]).wait()

        fire_gather(0, 0)

        def step(k, _):
            for b in range(2):
                ci = 2 * k + b
                fire_gather(ci + 1, 1 - b)
                wait_gather(b)

                @pl.when(k > 0)
                def _():
                    wait_scatter(b)

                compute(ci, b)
            return 0

        lax.fori_loop(0, (NCHUNK - 1) // 2, step, 0)
        last = NCHUNK - 1
        wait_gather(0)
        wait_scatter(0)
        compute(last, 0)
        wait_scatter(1)
        wait_scatter(0)

        plsc.subcore_barrier()
        pltpu.sync_copy(acc.at[pl.ds(s * ROWS, ROWS)],
                        out_hbm.at[c, pl.ds(s * ROWS, ROWS)])

    return pl.kernel(
        body,
        out_type=jax.ShapeDtypeStruct((2, NPAD, DT), _f32),
        mesh=mesh,
        compiler_params=pltpu.CompilerParams(
            use_tc_tiling_on_sc=False, needs_layout_passes=False),
        scratch_types=[
            pltpu.VMEM((NCHUNK, CHUNK), jnp.int32),
            pltpu.VMEM((NCHUNK, CHUNK), jnp.int32),
            pltpu.VMEM((2, CHUNK, DT), _f32),
            pltpu.VMEM((2, CHUNK, 16), _f32),
            pltpu.VMEM((2, CHUNK, DT), _f32),
            pltpu.VMEM_SHARED((NPAD, DT), _f32),
            pltpu.SemaphoreType.DMA,
            pltpu.SemaphoreType.DMA,
            pltpu.SemaphoreType.DMA,
            pltpu.SemaphoreType.DMA,
        ],
    )


_edge1 = _make_edge_sc(80, 64, 8)
_edge2 = _make_edge_sc(32, 16, 1)


_M64 = np.zeros((80, 64), np.float32)
_M64[:64, :] = np.eye(64, dtype=np.float32)
_B8F = np.zeros((80, 64), np.float32)
for _h in range(8):
    _B8F[64 + _h, 8 * _h:8 * _h + 8] = 1.0
_M16A = np.zeros((32, 16), np.float32)
_M16A[:16, :] = np.eye(16, dtype=np.float32)
_M16B = np.zeros((32, 16), np.float32)
_M16B[16:, :] = np.eye(16, dtype=np.float32)


def kernel(x, edge_index, W1, att1_src, att1_dst, b1, W2, att2_src, att2_dst, b2):
    src = edge_index[0].reshape(NW, NCHUNK, CHUNK)
    dst = edge_index[1].reshape(NW, NCHUNK, CHUNK)

    W1r = W1.reshape(D_FEAT, HEADS, HID)
    A1s = jnp.einsum('fhc,hc->fh', W1r, att1_src)
    A1d = jnp.einsum('fhc,hc->fh', W1r, att1_dst)
    z8 = jnp.zeros((D_FEAT, 8), _f32)
    W1T = jnp.concatenate([W1, A1s, z8], axis=1)
    W1A = jnp.concatenate([A1d, z8], axis=1)
    A2s = W2 @ att2_src[0]
    A2d = W2 @ att2_dst[0]
    W2T = jnp.concatenate([W2, jnp.tile(A2s[:, None], (1, 16))], axis=1)
    W2A = jnp.tile(A2d[:, None], (1, 16))

    bn = 1000
    grid = N_NODES // bn

    T1, AD1 = pl.pallas_call(
        _dense1_body,
        grid=(grid,),
        in_specs=[
            pl.BlockSpec((bn, D_FEAT), lambda i: (i, 0)),
            pl.BlockSpec((D_FEAT, 80), lambda i: (0, 0)),
            pl.BlockSpec((D_FEAT, 16), lambda i: (0, 0)),
        ],
        out_specs=[
            pl.BlockSpec((bn, 80), lambda i: (i, 0)),
            pl.BlockSpec((bn, 16), lambda i: (i, 0)),
        ],
        out_shape=[
            jax.ShapeDtypeStruct((N_NODES, 80), _f32),
            jax.ShapeDtypeStruct((N_NODES, 16), _f32),
        ],
    )(x, W1T, W1A)

    z1 = jnp.zeros((NPAD, 80), _f32)
    P1 = _edge1(T1, AD1, src, dst, z1)

    T2, AD2 = pl.pallas_call(
        _mid_body,
        grid=(grid,),
        in_specs=[
            pl.BlockSpec((2, bn, 80), lambda i: (0, i, 0)),
            pl.BlockSpec((1, 64), lambda i: (0, 0)),
            pl.BlockSpec((80, 64), lambda i: (0, 0)),
            pl.BlockSpec((80, 64), lambda i: (0, 0)),
            pl.BlockSpec((64, 32), lambda i: (0, 0)),
            pl.BlockSpec((64, 16), lambda i: (0, 0)),
        ],
        out_specs=[
            pl.BlockSpec((bn, 32), lambda i: (i, 0)),
            pl.BlockSpec((bn, 16), lambda i: (i, 0)),
        ],
        out_shape=[
            jax.ShapeDtypeStruct((N_NODES, 32), _f32),
            jax.ShapeDtypeStruct((N_NODES, 16), _f32),
        ],
    )(P1, b1[None, :], jnp.asarray(_M64), jnp.asarray(_B8F), W2T, W2A)

    z2 = jnp.zeros((NPAD, 32), _f32)
    P2 = _edge2(T2, AD2, src, dst, z2)

    out = pl.pallas_call(
        _final_body,
        grid=(grid,),
        in_specs=[
            pl.BlockSpec((2, bn, 32), lambda i: (0, i, 0)),
            pl.BlockSpec((1, 16), lambda i: (0, 0)),
            pl.BlockSpec((32, 16), lambda i: (0, 0)),
            pl.BlockSpec((32, 16), lambda i: (0, 0)),
        ],
        out_specs=pl.BlockSpec((bn, 16), lambda i: (i, 0)),
        out_shape=jax.ShapeDtypeStruct((N_NODES, 16), _f32),
    )(P2, b2[None, :], jnp.asarray(_M16A), jnp.asarray(_M16B))

    return out


# speedup vs baseline: 179.3418x; 2.1976x over previous
import functools

import jax
import jax.numpy as jnp
import numpy as np
from jax import lax
from jax.experimental import pallas as pl
from jax.experimental.pallas import tpu as pltpu
from jax.experimental.pallas import tpu_sc as plsc

N_NODES = 10000
N_EDGES = 320000
D_FEAT = 128
HID = 8
HEADS = 8
N_CLASSES = 16

NW = 32
EPW = N_EDGES // NW
CHUNK = 80
NCHUNK = EPW // CHUNK
NPAD = 10240
ROWS = NPAD // 16

_f32 = jnp.float32


def _dense1_body(x_ref, wt_ref, wa_ref, t1_ref, ad_ref):
    xb = x_ref[...]
    t1_ref[...] = jnp.dot(xb, wt_ref[...], preferred_element_type=_f32)
    ad_ref[...] = jnp.dot(xb, wa_ref[...], preferred_element_type=_f32)


def _mid_body(p_ref, b1_ref, m64_ref, b8_ref, w2t_ref, w2a_ref, t2_ref, ad2_ref):
    p = p_ref[...]
    m = p[0] + p[1]
    num = jnp.dot(m, m64_ref[...], preferred_element_type=_f32)
    den = jnp.dot(m, b8_ref[...], preferred_element_type=_f32)
    h = num / (den + 1e-16) + b1_ref[...]
    h = jnp.where(h > 0, h, jnp.exp(h) - 1.0)
    t2_ref[...] = jnp.dot(h, w2t_ref[...], preferred_element_type=_f32)
    ad2_ref[...] = jnp.dot(h, w2a_ref[...], preferred_element_type=_f32)


def _final_body(p_ref, b2_ref, ma_ref, mb_ref, o_ref):
    p = p_ref[...]
    m = p[0] + p[1]
    num = jnp.dot(m, ma_ref[...], preferred_element_type=_f32)
    den = jnp.dot(m, mb_ref[...], preferred_element_type=_f32)
    o = num / (den + 1e-16) + b2_ref[...]
    mx = jnp.max(o, axis=1, keepdims=True)
    e = jnp.exp(o - mx)
    s = jnp.sum(e, axis=1, keepdims=True)
    o_ref[...] = (o - mx) - jnp.log(s)


def _make_edge_sc(DT, DM, nheads):
    mesh = plsc.VectorSubcoreMesh(core_axis_name="c", subcore_axis_name="s")

    def body(t_hbm, ad_hbm, src_hbm, dst_hbm, z_hbm, out_hbm,
             siv, div, G, D, S, acc, gs0, gs1, ss0, ss1):
        c = lax.axis_index("c")
        s = lax.axis_index("s")
        wid = s * 2 + c
        pltpu.sync_copy(z_hbm.at[pl.ds(s * ROWS, ROWS)],
                        acc.at[pl.ds(s * ROWS, ROWS)])
        pltpu.sync_copy(src_hbm.at[wid], siv)
        pltpu.sync_copy(dst_hbm.at[wid], div)
        plsc.subcore_barrier()

        lane = lax.iota(jnp.int32, 16)
        pats = [jnp.where(lane >= 8, jnp.int32(2 * j + 1), jnp.int32(2 * j))
                for j in range(DM // 16)]
        gsems = (gs0, gs1)
        ssems = (ss0, ss1)

        def fire_gather(ci, b):
            pltpu.async_copy(t_hbm.at[siv.at[ci]], G.at[b], gsems[b])
            pltpu.async_copy(ad_hbm.at[div.at[ci]], D.at[b], gsems[b])

        def wait_gather(b):
            pltpu.make_async_copy(t_hbm.at[siv.at[0]], G.at[b], gsems[b]).wait()
            pltpu.make_async_copy(ad_hbm.at[div.at[0]], D.at[b], gsems[b]).wait()

        gdn = lax.GatherDimensionNumbers(
            offset_dims=(), collapsed_slice_dims=(0,), start_index_map=(0,))

        def take16(v, idx):
            return lax.gather(v, idx[:, None], dimension_numbers=gdn,
                              slice_sizes=(1,),
                              mode=lax.GatherScatterMode.PROMISE_IN_BOUNDS)

        def compute(ci, b):
            Gb = G.at[b]
            Db = D.at[b]
            Sb = S.at[b]

            @plsc.parallel_loop(0, CHUNK, unroll=4)
            def edge(i):
                ga = Gb[i, pl.ds(DM, 16)]
                da = Db[i, pl.ds(0, 16)]
                t = ga + da
                w = jnp.exp(jnp.maximum(t, 0.2 * t))
                if nheads == 8:
                    w = jnp.where(lane < 8, w, 0.0)
                Sb[i, pl.ds(DM, 16)] = w
                for j in range(DM // 16):
                    if nheads == 8:
                        cj = take16(w, pats[j])
                    else:
                        cj = w
                    Sb[i, pl.ds(16 * j, 16)] = cj * Gb[i, pl.ds(16 * j, 16)]

            pltpu.async_copy(S.at[b], acc.at[div.at[ci]], ssems[b], add=True)

        def wait_scatter(b):
            pltpu.make_async_copy(z_hbm.at[pl.ds(0, CHUNK)], S.at[b],
                                  ssems[b]).wait()

        fire_gather(0, 0)

        def step(k, _):
            for b in range(2):
                ci = 2 * k + b
                fire_gather(ci + 1, 1 - b)
                wait_gather(b)

                @pl.when(k > 0)
                def _():
                    wait_scatter(b)

                compute(ci, b)
            return 0

        lax.fori_loop(0, (NCHUNK - 1) // 2, step, 0)
        last = NCHUNK - 1
        wait_gather(0)
        wait_scatter(0)
        compute(last, 0)
        wait_scatter(1)
        wait_scatter(0)

        plsc.subcore_barrier()
        pltpu.sync_copy(acc.at[pl.ds(s * ROWS, ROWS)],
                        out_hbm.at[c, pl.ds(s * ROWS, ROWS)])

    return pl.kernel(
        body,
        out_type=jax.ShapeDtypeStruct((2, NPAD, DT), _f32),
        mesh=mesh,
        compiler_params=pltpu.CompilerParams(
            use_tc_tiling_on_sc=False, needs_layout_passes=False),
        scratch_types=[
            pltpu.VMEM((NCHUNK, CHUNK), jnp.int32),
            pltpu.VMEM((NCHUNK, CHUNK), jnp.int32),
            pltpu.VMEM((2, CHUNK, DT), _f32),
            pltpu.VMEM((2, CHUNK, 16), _f32),
            pltpu.VMEM((2, CHUNK, DT), _f32),
            pltpu.VMEM_SHARED((NPAD, DT), _f32),
            pltpu.SemaphoreType.DMA,
            pltpu.SemaphoreType.DMA,
            pltpu.SemaphoreType.DMA,
            pltpu.SemaphoreType.DMA,
        ],
    )


_edge1 = _make_edge_sc(80, 64, 8)
_edge2 = _make_edge_sc(32, 16, 1)


_M64 = np.zeros((80, 64), np.float32)
_M64[:64, :] = np.eye(64, dtype=np.float32)
_B8F = np.zeros((80, 64), np.float32)
for _h in range(8):
    _B8F[64 + _h, 8 * _h:8 * _h + 8] = 1.0
_M16A = np.zeros((32, 16), np.float32)
_M16A[:16, :] = np.eye(16, dtype=np.float32)
_M16B = np.zeros((32, 16), np.float32)
_M16B[16:, :] = np.eye(16, dtype=np.float32)


def kernel(x, edge_index, W1, att1_src, att1_dst, b1, W2, att2_src, att2_dst, b2):
    src = edge_index[0].reshape(NW, NCHUNK, CHUNK)
    dst = edge_index[1].reshape(NW, NCHUNK, CHUNK)

    W1r = W1.reshape(D_FEAT, HEADS, HID)
    A1s = jnp.einsum('fhc,hc->fh', W1r, att1_src)
    A1d = jnp.einsum('fhc,hc->fh', W1r, att1_dst)
    z8 = jnp.zeros((D_FEAT, 8), _f32)
    W1T = jnp.concatenate([W1, A1s, z8], axis=1)
    W1A = jnp.concatenate([A1d, z8], axis=1)
    A2s = W2 @ att2_src[0]
    A2d = W2 @ att2_dst[0]
    W2T = jnp.concatenate([W2, jnp.tile(A2s[:, None], (1, 16))], axis=1)
    W2A = jnp.tile(A2d[:, None], (1, 16))

    bn = 1000
    grid = N_NODES // bn

    T1, AD1 = pl.pallas_call(
        _dense1_body,
        grid=(grid,),
        in_specs=[
            pl.BlockSpec((bn, D_FEAT), lambda i: (i, 0)),
            pl.BlockSpec((D_FEAT, 80), lambda i: (0, 0)),
            pl.BlockSpec((D_FEAT, 16), lambda i: (0, 0)),
        ],
        out_specs=[
            pl.BlockSpec((bn, 80), lambda i: (i, 0)),
            pl.BlockSpec((bn, 16), lambda i: (i, 0)),
        ],
        out_shape=[
            jax.ShapeDtypeStruct((N_NODES, 80), _f32),
            jax.ShapeDtypeStruct((N_NODES, 16), _f32),
        ],
    )(x, W1T, W1A)

    z1 = jnp.zeros((NPAD, 80), _f32)
    P1 = _edge1(T1, AD1, src, dst, z1)

    T2, AD2 = pl.pallas_call(
        _mid_body,
        grid=(grid,),
        in_specs=[
            pl.BlockSpec((2, bn, 80), lambda i: (0, i, 0)),
            pl.BlockSpec((1, 64), lambda i: (0, 0)),
            pl.BlockSpec((80, 64), lambda i: (0, 0)),
            pl.BlockSpec((80, 64), lambda i: (0, 0)),
            pl.BlockSpec((64, 32), lambda i: (0, 0)),
            pl.BlockSpec((64, 16), lambda i: (0, 0)),
        ],
        out_specs=[
            pl.BlockSpec((bn, 32), lambda i: (i, 0)),
            pl.BlockSpec((bn, 16), lambda i: (i, 0)),
        ],
        out_shape=[
            jax.ShapeDtypeStruct((N_NODES, 32), _f32),
            jax.ShapeDtypeStruct((N_NODES, 16), _f32),
        ],
    )(P1, b1[None, :], jnp.asarray(_M64), jnp.asarray(_B8F), W2T, W2A)

    z2 = jnp.zeros((NPAD, 32), _f32)
    P2 = _edge2(T2, AD2, src, dst, z2)

    out = pl.pallas_call(
        _final_body,
        grid=(grid,),
        in_specs=[
            pl.BlockSpec((2, bn, 32), lambda i: (0, i, 0)),
            pl.BlockSpec((1, 16), lambda i: (0, 0)),
            pl.BlockSpec((32, 16), lambda i: (0, 0)),
            pl.BlockSpec((32, 16), lambda i: (0, 0)),
        ],
        out_specs=pl.BlockSpec((bn, 16), lambda i: (i, 0)),
        out_shape=jax.ShapeDtypeStruct((N_NODES, 16), _f32),
    )(P2, b2[None, :], jnp.asarray(_M16A), jnp.asarray(_M16B))

    return out


# speedup vs baseline: 183.6131x; 1.0238x over previous
import functools

import jax
import jax.numpy as jnp
import numpy as np
from jax import lax
from jax.experimental import pallas as pl
from jax.experimental.pallas import tpu as pltpu
from jax.experimental.pallas import tpu_sc as plsc

N_NODES = 10000
N_EDGES = 320000
D_FEAT = 128
HID = 8
HEADS = 8
N_CLASSES = 16

NW = 32
EPW = N_EDGES // NW
CHUNK = 100
NCHUNK = EPW // CHUNK
NPAD = 10240
ROWS = NPAD // 16

_f32 = jnp.float32


def _dense1_body(x_ref, wt_ref, wa_ref, t1_ref, ad_ref):
    xb = x_ref[...]
    t1_ref[...] = jnp.dot(xb, wt_ref[...], preferred_element_type=_f32)
    ad_ref[...] = jnp.dot(xb, wa_ref[...], preferred_element_type=_f32)


def _mid_body(p_ref, b1_ref, m64_ref, b8_ref, w2t_ref, w2a_ref, t2_ref, ad2_ref):
    p = p_ref[...]
    m = p[0] + p[1]
    num = jnp.dot(m, m64_ref[...], preferred_element_type=_f32)
    den = jnp.dot(m, b8_ref[...], preferred_element_type=_f32)
    h = num / (den + 1e-16) + b1_ref[...]
    h = jnp.where(h > 0, h, jnp.exp(h) - 1.0)
    t2_ref[...] = jnp.dot(h, w2t_ref[...], preferred_element_type=_f32)
    ad2_ref[...] = jnp.dot(h, w2a_ref[...], preferred_element_type=_f32)


def _final_body(p_ref, b2_ref, ma_ref, mb_ref, o_ref):
    p = p_ref[...]
    m = p[0] + p[1]
    num = jnp.dot(m, ma_ref[...], preferred_element_type=_f32)
    den = jnp.dot(m, mb_ref[...], preferred_element_type=_f32)
    o = num / (den + 1e-16) + b2_ref[...]
    mx = jnp.max(o, axis=1, keepdims=True)
    e = jnp.exp(o - mx)
    s = jnp.sum(e, axis=1, keepdims=True)
    o_ref[...] = (o - mx) - jnp.log(s)


def _make_edge_sc(DT, DM, nheads):
    mesh = plsc.VectorSubcoreMesh(core_axis_name="c", subcore_axis_name="s")

    def body(t_hbm, ad_hbm, src_hbm, dst_hbm, out_hbm,
             siv, div, G, D, S, acc, gs0, gs1, ss0, ss1):
        c = lax.axis_index("c")
        s = lax.axis_index("s")
        wid = s * 2 + c
        @plsc.parallel_loop(0, 80, unroll=4)
        def zrow(i):
            for j in range(DT // 16):
                S[0, i, pl.ds(16 * j, 16)] = jnp.zeros((16,), _f32)

        for r in range(ROWS // 80):
            pltpu.sync_copy(S.at[0].at[pl.ds(0, 80)],
                            acc.at[pl.ds(s * ROWS + r * 80, 80)])
        pltpu.sync_copy(src_hbm.at[wid], siv)
        pltpu.sync_copy(dst_hbm.at[wid], div)
        plsc.subcore_barrier()

        lane = lax.iota(jnp.int32, 16)
        pats = [jnp.where(lane >= 8, jnp.int32(2 * j + 1), jnp.int32(2 * j))
                for j in range(DM // 16)]
        gsems = (gs0, gs1)
        ssems = (ss0, ss1)

        def fire_gather(ci, b):
            pltpu.async_copy(t_hbm.at[siv.at[ci]], G.at[b], gsems[b])
            pltpu.async_copy(ad_hbm.at[div.at[ci]], D.at[b], gsems[b])

        def wait_gather(b):
            pltpu.make_async_copy(t_hbm.at[siv.at[0]], G.at[b], gsems[b]).wait()
            pltpu.make_async_copy(ad_hbm.at[div.at[0]], D.at[b], gsems[b]).wait()

        gdn = lax.GatherDimensionNumbers(
            offset_dims=(), collapsed_slice_dims=(0,), start_index_map=(0,))

        def take16(v, idx):
            return lax.gather(v, idx[:, None], dimension_numbers=gdn,
                              slice_sizes=(1,),
                              mode=lax.GatherScatterMode.PROMISE_IN_BOUNDS)

        def compute(ci, b):
            Gb = G.at[b]
            Db = D.at[b]
            Sb = S.at[b]

            @plsc.parallel_loop(0, CHUNK, unroll=8)
            def edge(i):
                ga = Gb[i, pl.ds(DM, 16)]
                da = Db[i, pl.ds(0, 16)]
                t = ga + da
                w = jnp.exp(jnp.maximum(t, 0.2 * t))
                if nheads == 8:
                    w = jnp.where(lane < 8, w, 0.0)
                Sb[i, pl.ds(DM, 16)] = w
                for j in range(DM // 16):
                    if nheads == 8:
                        cj = take16(w, pats[j])
                    else:
                        cj = w
                    Sb[i, pl.ds(16 * j, 16)] = cj * Gb[i, pl.ds(16 * j, 16)]

            pltpu.async_copy(S.at[b], acc.at[div.at[ci]], ssems[b], add=True)

        def wait_scatter(b):
            pltpu.make_async_copy(t_hbm.at[pl.ds(0, CHUNK)], S.at[b],
                                  ssems[b]).wait()

        fire_gather(0, 0)

        def step(k, _):
            for b in range(2):
                ci = 2 * k + b

                @pl.when(ci + 1 < NCHUNK)
                def _():
                    fire_gather(ci + 1, 1 - b)

                wait_gather(b)

                @pl.when(k > 0)
                def _():
                    wait_scatter(b)

                compute(ci, b)
            return 0

        lax.fori_loop(0, NCHUNK // 2, step, 0)
        if NCHUNK % 2 == 1:
            wait_gather(0)
            wait_scatter(0)
            compute(NCHUNK - 1, 0)
        wait_scatter(1)
        wait_scatter(0)

        plsc.subcore_barrier()
        pltpu.sync_copy(acc.at[pl.ds(s * ROWS, ROWS)],
                        out_hbm.at[c, pl.ds(s * ROWS, ROWS)])

    return pl.kernel(
        body,
        out_type=jax.ShapeDtypeStruct((2, NPAD, DT), _f32),
        mesh=mesh,
        compiler_params=pltpu.CompilerParams(
            use_tc_tiling_on_sc=False, needs_layout_passes=False),
        scratch_types=[
            pltpu.VMEM((NCHUNK, CHUNK), jnp.int32),
            pltpu.VMEM((NCHUNK, CHUNK), jnp.int32),
            pltpu.VMEM((2, CHUNK, DT), _f32),
            pltpu.VMEM((2, CHUNK, 16), _f32),
            pltpu.VMEM((2, CHUNK, DT), _f32),
            pltpu.VMEM_SHARED((NPAD, DT), _f32),
            pltpu.SemaphoreType.DMA,
            pltpu.SemaphoreType.DMA,
            pltpu.SemaphoreType.DMA,
            pltpu.SemaphoreType.DMA,
        ],
    )


_edge1 = _make_edge_sc(80, 64, 8)
_edge2 = _make_edge_sc(32, 16, 1)


_M64 = np.zeros((80, 64), np.float32)
_M64[:64, :] = np.eye(64, dtype=np.float32)
_B8F = np.zeros((80, 64), np.float32)
for _h in range(8):
    _B8F[64 + _h, 8 * _h:8 * _h + 8] = 1.0
_M16A = np.zeros((32, 16), np.float32)
_M16A[:16, :] = np.eye(16, dtype=np.float32)
_M16B = np.zeros((32, 16), np.float32)
_M16B[16:, :] = np.eye(16, dtype=np.float32)


def kernel(x, edge_index, W1, att1_src, att1_dst, b1, W2, att2_src, att2_dst, b2):
    src = edge_index[0].reshape(NW, NCHUNK, CHUNK)
    dst = edge_index[1].reshape(NW, NCHUNK, CHUNK)

    W1r = W1.reshape(D_FEAT, HEADS, HID)
    A1s = jnp.einsum('fhc,hc->fh', W1r, att1_src)
    A1d = jnp.einsum('fhc,hc->fh', W1r, att1_dst)
    z8 = jnp.zeros((D_FEAT, 8), _f32)
    W1T = jnp.concatenate([W1, A1s, z8], axis=1)
    W1A = jnp.concatenate([A1d, z8], axis=1)
    A2s = W2 @ att2_src[0]
    A2d = W2 @ att2_dst[0]
    W2T = jnp.concatenate([W2, jnp.tile(A2s[:, None], (1, 16))], axis=1)
    W2A = jnp.tile(A2d[:, None], (1, 16))

    bn = 1000
    grid = N_NODES // bn

    T1, AD1 = pl.pallas_call(
        _dense1_body,
        grid=(grid,),
        in_specs=[
            pl.BlockSpec((bn, D_FEAT), lambda i: (i, 0)),
            pl.BlockSpec((D_FEAT, 80), lambda i: (0, 0)),
            pl.BlockSpec((D_FEAT, 16), lambda i: (0, 0)),
        ],
        out_specs=[
            pl.BlockSpec((bn, 80), lambda i: (i, 0)),
            pl.BlockSpec((bn, 16), lambda i: (i, 0)),
        ],
        out_shape=[
            jax.ShapeDtypeStruct((N_NODES, 80), _f32),
            jax.ShapeDtypeStruct((N_NODES, 16), _f32),
        ],
    )(x, W1T, W1A)

    P1 = _edge1(T1, AD1, src, dst)

    T2, AD2 = pl.pallas_call(
        _mid_body,
        grid=(grid,),
        in_specs=[
            pl.BlockSpec((2, bn, 80), lambda i: (0, i, 0)),
            pl.BlockSpec((1, 64), lambda i: (0, 0)),
            pl.BlockSpec((80, 64), lambda i: (0, 0)),
            pl.BlockSpec((80, 64), lambda i: (0, 0)),
            pl.BlockSpec((64, 32), lambda i: (0, 0)),
            pl.BlockSpec((64, 16), lambda i: (0, 0)),
        ],
        out_specs=[
            pl.BlockSpec((bn, 32), lambda i: (i, 0)),
            pl.BlockSpec((bn, 16), lambda i: (i, 0)),
        ],
        out_shape=[
            jax.ShapeDtypeStruct((N_NODES, 32), _f32),
            jax.ShapeDtypeStruct((N_NODES, 16), _f32),
        ],
    )(P1, b1[None, :], jnp.asarray(_M64), jnp.asarray(_B8F), W2T, W2A)

    P2 = _edge2(T2, AD2, src, dst)

    out = pl.pallas_call(
        _final_body,
        grid=(grid,),
        in_specs=[
            pl.BlockSpec((2, bn, 32), lambda i: (0, i, 0)),
            pl.BlockSpec((1, 16), lambda i: (0, 0)),
            pl.BlockSpec((32, 16), lambda i: (0, 0)),
            pl.BlockSpec((32, 16), lambda i: (0, 0)),
        ],
        out_specs=pl.BlockSpec((bn, 16), lambda i: (i, 0)),
        out_shape=jax.ShapeDtypeStruct((N_NODES, 16), _f32),
    )(P2, b2[None, :], jnp.asarray(_M16A), jnp.asarray(_M16B))

    return out


# speedup vs baseline: 204.5824x; 1.1142x over previous
import functools

import jax
import jax.numpy as jnp
import numpy as np
from jax import lax
from jax.experimental import pallas as pl
from jax.experimental.pallas import tpu as pltpu
from jax.experimental.pallas import tpu_sc as plsc

N_NODES = 10000
N_EDGES = 320000
D_FEAT = 128
HID = 8
HEADS = 8
N_CLASSES = 16

NW = 32
EPW = N_EDGES // NW
CHUNK = 100
NCHUNK = EPW // CHUNK
NPAD = 10240
ROWS = NPAD // 16

_f32 = jnp.float32


def _dense1_body(x_ref, wt_ref, wa_ref, t1_ref, ad_ref):
    xb = x_ref[...]
    t1_ref[...] = jnp.dot(xb, wt_ref[...], preferred_element_type=_f32)
    ad_ref[...] = jnp.dot(xb, wa_ref[...], preferred_element_type=_f32)


def _mid_body(p_ref, b1_ref, m64_ref, b8_ref, w2t_ref, w2a_ref, t2_ref, ad2_ref):
    p = p_ref[...]
    m = p[0] + p[1]
    num = jnp.dot(m, m64_ref[...], preferred_element_type=_f32)
    den = jnp.dot(m, b8_ref[...], preferred_element_type=_f32)
    h = num / (den + 1e-16) + b1_ref[...]
    h = jnp.where(h > 0, h, jnp.exp(h) - 1.0)
    t2_ref[...] = jnp.dot(h, w2t_ref[...], preferred_element_type=_f32)
    ad2_ref[...] = jnp.dot(h, w2a_ref[...], preferred_element_type=_f32)


def _final_body(p_ref, b2_ref, ma_ref, mb_ref, o_ref):
    p = p_ref[...]
    m = p[0] + p[1]
    num = jnp.dot(m, ma_ref[...], preferred_element_type=_f32)
    den = jnp.dot(m, mb_ref[...], preferred_element_type=_f32)
    o = num / (den + 1e-16) + b2_ref[...]
    mx = jnp.max(o, axis=1, keepdims=True)
    e = jnp.exp(o - mx)
    s = jnp.sum(e, axis=1, keepdims=True)
    o_ref[...] = (o - mx) - jnp.log(s)


def _make_edge_sc(DT, DM, nheads):
    mesh = plsc.VectorSubcoreMesh(core_axis_name="c", subcore_axis_name="s")

    def body(t_hbm, ad_hbm, src_hbm, dst_hbm, out_hbm,
             siv, div, G, D, S, acc, gs0, gs1, gs2, ss0, ss1, ss2):
        c = lax.axis_index("c")
        s = lax.axis_index("s")
        wid = s * 2 + c
        @plsc.parallel_loop(0, 80, unroll=4)
        def zrow(i):
            for j in range(DT // 16):
                S[0, i, pl.ds(16 * j, 16)] = jnp.zeros((16,), _f32)

        for r in range(ROWS // 80):
            pltpu.sync_copy(S.at[0].at[pl.ds(0, 80)],
                            acc.at[pl.ds(s * ROWS + r * 80, 80)])
        pltpu.sync_copy(src_hbm.at[wid], siv)
        pltpu.sync_copy(dst_hbm.at[wid], div)
        plsc.subcore_barrier()

        lane = lax.iota(jnp.int32, 16)
        pats = [jnp.where(lane >= 8, jnp.int32(2 * j + 1), jnp.int32(2 * j))
                for j in range(DM // 16)]
        gsems = (gs0, gs1, gs2)
        ssems = (ss0, ss1, ss2)

        def fire_gather(ci, b):
            pltpu.async_copy(t_hbm.at[siv.at[ci]], G.at[b], gsems[b])
            pltpu.async_copy(ad_hbm.at[div.at[ci]], D.at[b], gsems[b])

        def wait_gather(b):
            pltpu.make_async_copy(t_hbm.at[siv.at[0]], G.at[b], gsems[b]).wait()
            pltpu.make_async_copy(ad_hbm.at[div.at[0]], D.at[b], gsems[b]).wait()

        gdn = lax.GatherDimensionNumbers(
            offset_dims=(), collapsed_slice_dims=(0,), start_index_map=(0,))

        def take16(v, idx):
            return lax.gather(v, idx[:, None], dimension_numbers=gdn,
                              slice_sizes=(1,),
                              mode=lax.GatherScatterMode.PROMISE_IN_BOUNDS)

        def compute(ci, b):
            Gb = G.at[b]
            Db = D.at[b]
            Sb = S.at[b]

            @plsc.parallel_loop(0, CHUNK, unroll=8)
            def edge(i):
                ga = Gb[i, pl.ds(DM, 16)]
                da = Db[i, pl.ds(0, 16)]
                t = ga + da
                w = jnp.exp(jnp.maximum(t, 0.2 * t))
                if nheads == 8:
                    w = jnp.where(lane < 8, w, 0.0)
                Sb[i, pl.ds(DM, 16)] = w
                for j in range(DM // 16):
                    if nheads == 8:
                        cj = take16(w, pats[j])
                    else:
                        cj = w
                    Sb[i, pl.ds(16 * j, 16)] = cj * Gb[i, pl.ds(16 * j, 16)]

            pltpu.async_copy(S.at[b], acc.at[div.at[ci]], ssems[b], add=True)

        def wait_scatter(b):
            pltpu.make_async_copy(t_hbm.at[pl.ds(0, CHUNK)], S.at[b],
                                  ssems[b]).wait()

        assert NCHUNK % 3 == 1
        fire_gather(0, 0)
        fire_gather(1, 1)

        def step(k, _):
            for b in range(3):
                ci = 3 * k + b

                @pl.when(ci + 2 < NCHUNK)
                def _():
                    fire_gather(ci + 2, (b + 2) % 3)

                wait_gather(b)

                @pl.when(k > 0)
                def _():
                    wait_scatter(b)

                compute(ci, b)
            return 0

        lax.fori_loop(0, NCHUNK // 3, step, 0)
        wait_gather(0)
        wait_scatter(0)
        compute(NCHUNK - 1, 0)
        wait_scatter(1)
        wait_scatter(2)
        wait_scatter(0)

        plsc.subcore_barrier()
        pltpu.sync_copy(acc.at[pl.ds(s * ROWS, ROWS)],
                        out_hbm.at[c, pl.ds(s * ROWS, ROWS)])

    return pl.kernel(
        body,
        out_type=jax.ShapeDtypeStruct((2, NPAD, DT), _f32),
        mesh=mesh,
        compiler_params=pltpu.CompilerParams(
            use_tc_tiling_on_sc=False, needs_layout_passes=False),
        scratch_types=[
            pltpu.VMEM((NCHUNK, CHUNK), jnp.int32),
            pltpu.VMEM((NCHUNK, CHUNK), jnp.int32),
            pltpu.VMEM((3, CHUNK, DT), _f32),
            pltpu.VMEM((3, CHUNK, 16), _f32),
            pltpu.VMEM((3, CHUNK, DT), _f32),
            pltpu.VMEM_SHARED((NPAD, DT), _f32),
            pltpu.SemaphoreType.DMA,
            pltpu.SemaphoreType.DMA,
            pltpu.SemaphoreType.DMA,
            pltpu.SemaphoreType.DMA,
            pltpu.SemaphoreType.DMA,
            pltpu.SemaphoreType.DMA,
        ],
    )


_edge1 = _make_edge_sc(80, 64, 8)
_edge2 = _make_edge_sc(32, 16, 1)


_M64 = np.zeros((80, 64), np.float32)
_M64[:64, :] = np.eye(64, dtype=np.float32)
_B8F = np.zeros((80, 64), np.float32)
for _h in range(8):
    _B8F[64 + _h, 8 * _h:8 * _h + 8] = 1.0
_M16A = np.zeros((32, 16), np.float32)
_M16A[:16, :] = np.eye(16, dtype=np.float32)
_M16B = np.zeros((32, 16), np.float32)
_M16B[16:, :] = np.eye(16, dtype=np.float32)


def kernel(x, edge_index, W1, att1_src, att1_dst, b1, W2, att2_src, att2_dst, b2):
    src = edge_index[0].reshape(NW, NCHUNK, CHUNK)
    dst = edge_index[1].reshape(NW, NCHUNK, CHUNK)

    W1r = W1.reshape(D_FEAT, HEADS, HID)
    A1s = jnp.einsum('fhc,hc->fh', W1r, att1_src)
    A1d = jnp.einsum('fhc,hc->fh', W1r, att1_dst)
    z8 = jnp.zeros((D_FEAT, 8), _f32)
    W1T = jnp.concatenate([W1, A1s, z8], axis=1)
    W1A = jnp.concatenate([A1d, z8], axis=1)
    A2s = W2 @ att2_src[0]
    A2d = W2 @ att2_dst[0]
    W2T = jnp.concatenate([W2, jnp.tile(A2s[:, None], (1, 16))], axis=1)
    W2A = jnp.tile(A2d[:, None], (1, 16))

    bn = 1000
    grid = N_NODES // bn

    T1, AD1 = pl.pallas_call(
        _dense1_body,
        grid=(grid,),
        in_specs=[
            pl.BlockSpec((bn, D_FEAT), lambda i: (i, 0)),
            pl.BlockSpec((D_FEAT, 80), lambda i: (0, 0)),
            pl.BlockSpec((D_FEAT, 16), lambda i: (0, 0)),
        ],
        out_specs=[
            pl.BlockSpec((bn, 80), lambda i: (i, 0)),
            pl.BlockSpec((bn, 16), lambda i: (i, 0)),
        ],
        out_shape=[
            jax.ShapeDtypeStruct((N_NODES, 80), _f32),
            jax.ShapeDtypeStruct((N_NODES, 16), _f32),
        ],
    )(x, W1T, W1A)

    P1 = _edge1(T1, AD1, src, dst)

    T2, AD2 = pl.pallas_call(
        _mid_body,
        grid=(grid,),
        in_specs=[
            pl.BlockSpec((2, bn, 80), lambda i: (0, i, 0)),
            pl.BlockSpec((1, 64), lambda i: (0, 0)),
            pl.BlockSpec((80, 64), lambda i: (0, 0)),
            pl.BlockSpec((80, 64), lambda i: (0, 0)),
            pl.BlockSpec((64, 32), lambda i: (0, 0)),
            pl.BlockSpec((64, 16), lambda i: (0, 0)),
        ],
        out_specs=[
            pl.BlockSpec((bn, 32), lambda i: (i, 0)),
            pl.BlockSpec((bn, 16), lambda i: (i, 0)),
        ],
        out_shape=[
            jax.ShapeDtypeStruct((N_NODES, 32), _f32),
            jax.ShapeDtypeStruct((N_NODES, 16), _f32),
        ],
    )(P1, b1[None, :], jnp.asarray(_M64), jnp.asarray(_B8F), W2T, W2A)

    P2 = _edge2(T2, AD2, src, dst)

    out = pl.pallas_call(
        _final_body,
        grid=(grid,),
        in_specs=[
            pl.BlockSpec((2, bn, 32), lambda i: (0, i, 0)),
            pl.BlockSpec((1, 16), lambda i: (0, 0)),
            pl.BlockSpec((32, 16), lambda i: (0, 0)),
            pl.BlockSpec((32, 16), lambda i: (0, 0)),
        ],
        out_specs=pl.BlockSpec((bn, 16), lambda i: (i, 0)),
        out_shape=jax.ShapeDtypeStruct((N_NODES, 16), _f32),
    )(P2, b2[None, :], jnp.asarray(_M16A), jnp.asarray(_M16B))

    return out


# speedup vs baseline: 215.5530x; 1.0536x over previous
import functools

import jax
import jax.numpy as jnp
import numpy as np
from jax import lax
from jax.experimental import pallas as pl
from jax.experimental.pallas import tpu as pltpu
from jax.experimental.pallas import tpu_sc as plsc

N_NODES = 10000
N_EDGES = 320000
D_FEAT = 128
HID = 8
HEADS = 8
N_CLASSES = 16

NW = 32
EPW = N_EDGES // NW
CHUNK = 100
NCHUNK = EPW // CHUNK
NPAD = 10240
ROWS = NPAD // 16

_f32 = jnp.float32


def _dense1_body(x_ref, wt_ref, wa_ref, t1_ref, ad_ref):
    xb = x_ref[...]
    t1_ref[...] = jnp.dot(xb, wt_ref[...], preferred_element_type=_f32)
    ad_ref[...] = jnp.dot(xb, wa_ref[...], preferred_element_type=_f32)


def _mid_body(p_ref, b1_ref, m64_ref, b8_ref, w2t_ref, w2a_ref, t2_ref, ad2_ref):
    p = p_ref[...]
    m = p[0] + p[1]
    num = jnp.dot(m, m64_ref[...], preferred_element_type=_f32)
    den = jnp.dot(m, b8_ref[...], preferred_element_type=_f32)
    h = num / (den + 1e-16) + b1_ref[...]
    h = jnp.where(h > 0, h, jnp.exp(h) - 1.0)
    t2_ref[...] = jnp.dot(h, w2t_ref[...], preferred_element_type=_f32)
    ad2_ref[...] = jnp.dot(h, w2a_ref[...], preferred_element_type=_f32)


def _final_body(p_ref, b2_ref, ma_ref, mb_ref, o_ref):
    p = p_ref[...]
    m = p[0] + p[1]
    num = jnp.dot(m, ma_ref[...], preferred_element_type=_f32)
    den = jnp.dot(m, mb_ref[...], preferred_element_type=_f32)
    o = num / (den + 1e-16) + b2_ref[...]
    mx = jnp.max(o, axis=1, keepdims=True)
    e = jnp.exp(o - mx)
    s = jnp.sum(e, axis=1, keepdims=True)
    o_ref[...] = (o - mx) - jnp.log(s)


def _make_edge_sc(DT, DM, nheads):
    mesh = plsc.VectorSubcoreMesh(core_axis_name="c", subcore_axis_name="s")

    def body(t_hbm, ad_hbm, src_hbm, dst_hbm, out_hbm,
             siv, div, G, D, S, acc, gs0, gs1, gs2, ss0, ss1, ss2):
        c = lax.axis_index("c")
        s = lax.axis_index("s")
        wid = s * 2 + c
        pltpu.sync_copy(src_hbm.at[wid], siv)
        pltpu.sync_copy(dst_hbm.at[wid], div)

        lane = lax.iota(jnp.int32, 16)
        pats = [jnp.where(lane >= 8, jnp.int32(2 * j + 1), jnp.int32(2 * j))
                for j in range(DM // 16)]
        gsems = (gs0, gs1, gs2)
        ssems = (ss0, ss1, ss2)

        def fire_gather(ci, b):
            pltpu.async_copy(t_hbm.at[siv.at[ci]], G.at[b], gsems[b])
            pltpu.async_copy(ad_hbm.at[div.at[ci]], D.at[b], gsems[b])

        def wait_gather(b):
            pltpu.make_async_copy(t_hbm.at[siv.at[0]], G.at[b], gsems[b]).wait()
            pltpu.make_async_copy(ad_hbm.at[div.at[0]], D.at[b], gsems[b]).wait()

        gdn = lax.GatherDimensionNumbers(
            offset_dims=(), collapsed_slice_dims=(0,), start_index_map=(0,))

        def take16(v, idx):
            return lax.gather(v, idx[:, None], dimension_numbers=gdn,
                              slice_sizes=(1,),
                              mode=lax.GatherScatterMode.PROMISE_IN_BOUNDS)

        def compute(ci, b):
            Gb = G.at[b]
            Db = D.at[b]
            Sb = S.at[b]

            @plsc.parallel_loop(0, CHUNK, unroll=8)
            def edge(i):
                ga = Gb[i, pl.ds(DM, 16)]
                da = Db[i, pl.ds(0, 16)]
                t = ga + da
                w = jnp.exp(jnp.maximum(t, 0.2 * t))
                if nheads == 8:
                    w = jnp.where(lane < 8, w, 0.0)
                Sb[i, pl.ds(DM, 16)] = w
                for j in range(DM // 16):
                    if nheads == 8:
                        cj = take16(w, pats[j])
                    else:
                        cj = w
                    Sb[i, pl.ds(16 * j, 16)] = cj * Gb[i, pl.ds(16 * j, 16)]

            pltpu.async_copy(S.at[b], acc.at[div.at[ci]], ssems[b], add=True)

        def wait_scatter(b):
            pltpu.make_async_copy(t_hbm.at[pl.ds(0, CHUNK)], S.at[b],
                                  ssems[b]).wait()

        assert NCHUNK % 3 == 1
        fire_gather(0, 0)
        fire_gather(1, 1)

        @plsc.parallel_loop(0, 80, unroll=4)
        def zrow(i):
            for j in range(DT // 16):
                S[2, i, pl.ds(16 * j, 16)] = jnp.zeros((16,), _f32)

        for r in range(ROWS // 80):
            pltpu.sync_copy(S.at[2].at[pl.ds(0, 80)],
                            acc.at[pl.ds(s * ROWS + r * 80, 80)])
        plsc.subcore_barrier()

        def step(k, _):
            for b in range(3):
                ci = 3 * k + b

                @pl.when(ci + 2 < NCHUNK)
                def _():
                    fire_gather(ci + 2, (b + 2) % 3)

                wait_gather(b)

                @pl.when(k > 0)
                def _():
                    wait_scatter(b)

                compute(ci, b)
            return 0

        lax.fori_loop(0, NCHUNK // 3, step, 0)
        wait_gather(0)
        wait_scatter(0)
        compute(NCHUNK - 1, 0)
        wait_scatter(1)
        wait_scatter(2)
        wait_scatter(0)

        plsc.subcore_barrier()
        pltpu.sync_copy(acc.at[pl.ds(s * ROWS, ROWS)],
                        out_hbm.at[c, pl.ds(s * ROWS, ROWS)])

    return pl.kernel(
        body,
        out_type=jax.ShapeDtypeStruct((2, NPAD, DT), _f32),
        mesh=mesh,
        compiler_params=pltpu.CompilerParams(
            use_tc_tiling_on_sc=False, needs_layout_passes=False),
        scratch_types=[
            pltpu.VMEM((NCHUNK, CHUNK), jnp.int32),
            pltpu.VMEM((NCHUNK, CHUNK), jnp.int32),
            pltpu.VMEM((3, CHUNK, DT), _f32),
            pltpu.VMEM((3, CHUNK, 16), _f32),
            pltpu.VMEM((3, CHUNK, DT), _f32),
            pltpu.VMEM_SHARED((NPAD, DT), _f32),
            pltpu.SemaphoreType.DMA,
            pltpu.SemaphoreType.DMA,
            pltpu.SemaphoreType.DMA,
            pltpu.SemaphoreType.DMA,
            pltpu.SemaphoreType.DMA,
            pltpu.SemaphoreType.DMA,
        ],
    )


_edge1 = _make_edge_sc(80, 64, 8)
_edge2 = _make_edge_sc(32, 16, 1)


_M64 = np.zeros((80, 64), np.float32)
_M64[:64, :] = np.eye(64, dtype=np.float32)
_B8F = np.zeros((80, 64), np.float32)
for _h in range(8):
    _B8F[64 + _h, 8 * _h:8 * _h + 8] = 1.0
_M16A = np.zeros((32, 16), np.float32)
_M16A[:16, :] = np.eye(16, dtype=np.float32)
_M16B = np.zeros((32, 16), np.float32)
_M16B[16:, :] = np.eye(16, dtype=np.float32)


def kernel(x, edge_index, W1, att1_src, att1_dst, b1, W2, att2_src, att2_dst, b2):
    src = edge_index[0].reshape(NW, NCHUNK, CHUNK)
    dst = edge_index[1].reshape(NW, NCHUNK, CHUNK)

    W1r = W1.reshape(D_FEAT, HEADS, HID)
    A1s = jnp.einsum('fhc,hc->fh', W1r, att1_src)
    A1d = jnp.einsum('fhc,hc->fh', W1r, att1_dst)
    z8 = jnp.zeros((D_FEAT, 8), _f32)
    W1T = jnp.concatenate([W1, A1s, z8], axis=1)
    W1A = jnp.concatenate([A1d, z8], axis=1)
    A2s = W2 @ att2_src[0]
    A2d = W2 @ att2_dst[0]
    W2T = jnp.concatenate([W2, jnp.tile(A2s[:, None], (1, 16))], axis=1)
    W2A = jnp.tile(A2d[:, None], (1, 16))

    bn = N_NODES
    grid = 1

    T1, AD1 = pl.pallas_call(
        _dense1_body,
        grid=(grid,),
        in_specs=[
            pl.BlockSpec((bn, D_FEAT), lambda i: (i, 0)),
            pl.BlockSpec((D_FEAT, 80), lambda i: (0, 0)),
            pl.BlockSpec((D_FEAT, 16), lambda i: (0, 0)),
        ],
        out_specs=[
            pl.BlockSpec((bn, 80), lambda i: (i, 0)),
            pl.BlockSpec((bn, 16), lambda i: (i, 0)),
        ],
        out_shape=[
            jax.ShapeDtypeStruct((N_NODES, 80), _f32),
            jax.ShapeDtypeStruct((N_NODES, 16), _f32),
        ],
    )(x, W1T, W1A)

    P1 = _edge1(T1, AD1, src, dst)

    T2, AD2 = pl.pallas_call(
        _mid_body,
        grid=(grid,),
        in_specs=[
            pl.BlockSpec((2, bn, 80), lambda i: (0, i, 0)),
            pl.BlockSpec((1, 64), lambda i: (0, 0)),
            pl.BlockSpec((80, 64), lambda i: (0, 0)),
            pl.BlockSpec((80, 64), lambda i: (0, 0)),
            pl.BlockSpec((64, 32), lambda i: (0, 0)),
            pl.BlockSpec((64, 16), lambda i: (0, 0)),
        ],
        out_specs=[
            pl.BlockSpec((bn, 32), lambda i: (i, 0)),
            pl.BlockSpec((bn, 16), lambda i: (i, 0)),
        ],
        out_shape=[
            jax.ShapeDtypeStruct((N_NODES, 32), _f32),
            jax.ShapeDtypeStruct((N_NODES, 16), _f32),
        ],
    )(P1, b1[None, :], jnp.asarray(_M64), jnp.asarray(_B8F), W2T, W2A)

    P2 = _edge2(T2, AD2, src, dst)

    out = pl.pallas_call(
        _final_body,
        grid=(grid,),
        in_specs=[
            pl.BlockSpec((2, bn, 32), lambda i: (0, i, 0)),
            pl.BlockSpec((1, 16), lambda i: (0, 0)),
            pl.BlockSpec((32, 16), lambda i: (0, 0)),
            pl.BlockSpec((32, 16), lambda i: (0, 0)),
        ],
        out_specs=pl.BlockSpec((bn, 16), lambda i: (i, 0)),
        out_shape=jax.ShapeDtypeStruct((N_NODES, 16), _f32),
    )(P2, b2[None, :], jnp.asarray(_M16A), jnp.asarray(_M16B))

    return out
